# baseline (speedup 1.0000x reference)
import functools

import jax
import jax.numpy as jnp
from jax import lax
from jax.experimental import pallas as pl
from jax.experimental.pallas import tpu as pltpu
from jax.experimental.pallas import tpu_sc as plsc

N = 10000
E = 320000
D = 128
HD = D // 2
NC = 2
NS = 16
NW = NC * NS
CH = 80
NCH = 125
EPW = NCH * CH
RPS = 624
TAIL = N - RPS * NS
K = 5

_MESH = plsc.VectorSubcoreMesh(
    core_axis_name="c", subcore_axis_name="s", num_cores=NC, num_subcores=NS
)


def _sc_body(with_deg, *refs):
    if with_deg:
        (table, src3, dst3, out0, out1, dego0, dego1,
         src_v, dst_v, rows_v, ones_v, zrow_v, zdeg_v,
         acc_sh, deg_sh, *sems) = refs
    else:
        (table, src3, dst3, out0, out1,
         src_v, dst_v, rows_v, zrow_v,
         acc_sh, *sems) = refs
    gsems = sems[:K]
    ssems = sems[K:2 * K]
    dsems = sems[2 * K:]

    c = lax.axis_index("c")
    s = lax.axis_index("s")
    wid = c * NS + s

    zeros16 = jnp.zeros((16,), jnp.float32)
    zeros32 = jnp.zeros((32,), jnp.bfloat16)
    start = pl.multiple_of(s * RPS, 16)

    def _zrow(i, _):
        for k in range(D // 32):
            zrow_v[i, pl.ds(k * 32, 32)] = zeros32
        return 0

    lax.fori_loop(0, 104, _zrow, 0)

    def _zacc(i, _):
        pltpu.sync_copy(zrow_v, acc_sh.at[pl.ds(pl.multiple_of(start + i * 104, 8), 104)])
        return 0

    lax.fori_loop(0, RPS // 104, _zacc, 0)

    @pl.when(s == NS - 1)
    def _():
        pltpu.sync_copy(zrow_v.at[pl.ds(0, TAIL)], acc_sh.at[pl.ds(RPS * NS, TAIL)])

    if with_deg:
        def _zdeg(i, _):
            zdeg_v[i] = zeros16
            return 0

        lax.fori_loop(0, 104, _zdeg, 0)

        ones16 = jnp.ones((16,), jnp.float32)

        def _ones(i, _):
            ones_v[i] = ones16
            return 0

        lax.fori_loop(0, CH, _ones, 0)

        def _zdacc(i, _):
            pltpu.sync_copy(
                zdeg_v, deg_sh.at[pl.ds(pl.multiple_of(start + i * 104, 8), 104)])
            return 0

        lax.fori_loop(0, RPS // 104, _zdacc, 0)

        @pl.when(s == NS - 1)
        def _():
            pltpu.sync_copy(zdeg_v.at[pl.ds(0, TAIL)],
                            deg_sh.at[pl.ds(RPS * NS, TAIL)])

    pltpu.sync_copy(src3.at[wid], src_v)
    pltpu.sync_copy(dst3.at[wid], dst_v)

    plsc.subcore_barrier()

    def _iter(it, _):
        base = it * K

        @pl.when(it > 0)
        def _():
            for k in range(K):
                pltpu.make_async_copy(
                    rows_v.at[k], acc_sh.at[dst_v.at[0]], ssems[k]).wait()
                if with_deg:
                    pltpu.make_async_copy(
                        ones_v, deg_sh.at[dst_v.at[0]], dsems[k]).wait()

        gd = [
            pltpu.async_copy(table.at[src_v.at[base + k]],
                             rows_v.at[k], gsems[k])
            for k in range(K)
        ]
        for k in range(K):
            gd[k].wait()
            pltpu.async_copy(
                rows_v.at[k], acc_sh.at[dst_v.at[base + k]], ssems[k],
                add=True)
            if with_deg:
                pltpu.async_copy(
                    ones_v, deg_sh.at[dst_v.at[base + k]], dsems[k],
                    add=True)
        return 0

    lax.fori_loop(0, NCH // K, _iter, 0)
    for k in range(K):
        pltpu.make_async_copy(
            rows_v.at[k], acc_sh.at[dst_v.at[0]], ssems[k]).wait()
        if with_deg:
            pltpu.make_async_copy(
                ones_v, deg_sh.at[dst_v.at[0]], dsems[k]).wait()

    plsc.subcore_barrier()

    def _share_copy(src_sh, dst_hbm):
        pltpu.sync_copy(src_sh.at[pl.ds(start, RPS)], dst_hbm.at[pl.ds(start, RPS)])

        @pl.when(s == NS - 1)
        def _():
            pltpu.sync_copy(src_sh.at[pl.ds(RPS * NS, TAIL)],
                            dst_hbm.at[pl.ds(RPS * NS, TAIL)])

    @pl.when(c == 0)
    def _():
        _share_copy(acc_sh, out0)
        if with_deg:
            _share_copy(deg_sh, dego0)

    @pl.when(c == 1)
    def _():
        _share_copy(acc_sh, out1)
        if with_deg:
            _share_copy(deg_sh, dego1)


def _make_sc(with_deg):
    f32 = jnp.float32
    bf16 = jnp.bfloat16
    outs = [jax.ShapeDtypeStruct((N, D), bf16), jax.ShapeDtypeStruct((N, D), bf16)]
    scratch = [
        pltpu.VMEM((NCH, CH), jnp.int32),
        pltpu.VMEM((NCH, CH), jnp.int32),
        pltpu.VMEM((K, CH, D), bf16),
    ]
    if with_deg:
        outs += [jax.ShapeDtypeStruct((N, 16), f32), jax.ShapeDtypeStruct((N, 16), f32)]
        scratch += [pltpu.VMEM((CH, 16), f32)]
    scratch += [pltpu.VMEM((104, D), bf16)]
    if with_deg:
        scratch += [pltpu.VMEM((104, 16), f32)]
    scratch += [pltpu.VMEM_SHARED((N, D), bf16)]
    if with_deg:
        scratch += [pltpu.VMEM_SHARED((N, 16), f32)]
    nsem = 3 * K if with_deg else 2 * K
    scratch += [pltpu.SemaphoreType.DMA] * nsem

    return pl.kernel(
        functools.partial(_sc_body, with_deg),
        out_type=tuple(outs),
        mesh=_MESH,
        scratch_types=scratch,
        compiler_params=pltpu.CompilerParams(use_tc_tiling_on_sc=False),
    )


_SC_L1 = _make_sc(True)
_SC_L2 = _make_sc(False)

_BLK = 1000


def _deg(dg0_ref, dg1_ref):
    dg = dg0_ref[:, 0:1] + dg1_ref[:, 0:1]
    return jnp.maximum(dg, 1.0)


def _tc_lin_r(x_ref, w_ref, b_ref, o_ref):
    dn = (((1,), (1,)), ((), ()))
    o_ref[...] = b_ref[...] + lax.dot_general(
        x_ref[...].astype(jnp.float32), w_ref[...], dn,
        preferred_element_type=jnp.float32)


def _tc_body1(xr_ref, p0_ref, p1_ref, dg0_ref, dg1_ref, wl_ref,
              ob_ref):
    agg = p0_ref[...].astype(jnp.float32) + p1_ref[...].astype(jnp.float32)
    mean = agg / _deg(dg0_ref, dg1_ref)
    dn = (((1,), (1,)), ((), ()))
    h = lax.dot_general(mean, wl_ref[...], dn, preferred_element_type=jnp.float32)
    h = h + xr_ref[...]
    ob_ref[...] = jax.nn.relu(h).astype(jnp.bfloat16)


def _tc_body2(hr_ref, q0_ref, q1_ref, dg0_ref, dg1_ref, wl_ref, o_ref):
    agg = q0_ref[...].astype(jnp.float32) + q1_ref[...].astype(jnp.float32)
    mean = agg / _deg(dg0_ref, dg1_ref)
    dn = (((1,), (1,)), ((), ()))
    h = lax.dot_general(mean, wl_ref[...], dn, preferred_element_type=jnp.float32)
    h = h + hr_ref[...]
    o_ref[...] = jax.nn.sigmoid(h)


_row = pl.BlockSpec((_BLK, D), lambda i: (i, 0))
_half = pl.BlockSpec((_BLK, HD), lambda i: (i, 0))
_dgs = pl.BlockSpec((_BLK, 16), lambda i: (i, 0))
_full = pl.BlockSpec((D, D), lambda i: (0, 0))
_bias = pl.BlockSpec((1, D), lambda i: (0, 0))

_rowb = pl.BlockSpec((_BLK, D), lambda i: (i, 0))

_TC_LIN = pl.pallas_call(
    _tc_lin_r,
    grid=(N // _BLK,),
    in_specs=[_rowb, _full, _bias],
    out_specs=_row,
    out_shape=jax.ShapeDtypeStruct((N, D), jnp.float32),
)

_TC_L1 = pl.pallas_call(
    _tc_body1,
    grid=(N // _BLK,),
    in_specs=[_row, _rowb, _rowb, _dgs, _dgs, _full],
    out_specs=_rowb,
    out_shape=jax.ShapeDtypeStruct((N, D), jnp.bfloat16),
)

_TC_L2 = pl.pallas_call(
    _tc_body2,
    grid=(N // _BLK,),
    in_specs=[_row, _rowb, _rowb, _dgs, _dgs, _full],
    out_specs=_row,
    out_shape=jax.ShapeDtypeStruct((N, D), jnp.float32),
)


def kernel(x, edge_index, W1_l, b1_l, W1_r, W2_l, b2_l, W2_r):
    src3 = edge_index[0].astype(jnp.int32).reshape(NW, NCH, CH)
    dst3 = edge_index[1].astype(jnp.int32).reshape(NW, NCH, CH)
    xb = x.astype(jnp.bfloat16)

    xr = _TC_LIN(xb, W1_r, b1_l.reshape(1, D))
    p0, p1, dg0, dg1 = _SC_L1(xb, src3, dst3)
    hb = _TC_L1(xr, p0, p1, dg0, dg1, W1_l)
    hr = _TC_LIN(hb, W2_r, b2_l.reshape(1, D))
    q0, q1 = _SC_L2(hb, src3, dst3)
    return _TC_L2(hr, q0, q1, dg0, dg1, W2_l)


# speedup vs baseline: 1.0391x; 1.0391x over previous
import functools

import jax
import jax.numpy as jnp
from jax import lax
from jax.experimental import pallas as pl
from jax.experimental.pallas import tpu as pltpu
from jax.experimental.pallas import tpu_sc as plsc

N = 10000
E = 320000
D = 128
HD = D // 2
NC = 2
NS = 16
NW = NC * NS
CH = 80
NCH = 125
EPW = NCH * CH
RPS = 624
TAIL = N - RPS * NS
K = 5

_MESH = plsc.VectorSubcoreMesh(
    core_axis_name="c", subcore_axis_name="s", num_cores=NC, num_subcores=NS
)


def _sc_body(with_deg, *refs):
    if with_deg:
        (table, src3, dst3, out0, out1, dego0, dego1,
         src_v, dst_v, rows_v, ones_v, zrow_v, zdeg_v,
         acc_sh, deg_sh, *sems) = refs
    else:
        (table, src3, dst3, out0, out1,
         src_v, dst_v, rows_v, zrow_v,
         acc_sh, *sems) = refs
    gsems = sems[:K]
    ssems = sems[K:2 * K]
    dsems = sems[2 * K:]

    c = lax.axis_index("c")
    s = lax.axis_index("s")
    wid = c * NS + s

    zeros16 = jnp.zeros((16,), jnp.float32)
    zeros32 = jnp.zeros((32,), jnp.bfloat16)
    start = pl.multiple_of(s * RPS, 16)

    def _zrow(i, _):
        for k in range(D // 32):
            zrow_v[i, pl.ds(k * 32, 32)] = zeros32
        return 0

    lax.fori_loop(0, 104, _zrow, 0)

    def _zacc(i, _):
        pltpu.sync_copy(zrow_v, acc_sh.at[pl.ds(pl.multiple_of(start + i * 104, 8), 104)])
        return 0

    lax.fori_loop(0, RPS // 104, _zacc, 0)

    @pl.when(s == NS - 1)
    def _():
        pltpu.sync_copy(zrow_v.at[pl.ds(0, TAIL)], acc_sh.at[pl.ds(RPS * NS, TAIL)])

    if with_deg:
        def _zdeg(i, _):
            zdeg_v[i] = zeros16
            return 0

        lax.fori_loop(0, 104, _zdeg, 0)

        ones16 = jnp.ones((16,), jnp.float32)

        def _ones(i, _):
            ones_v[i] = ones16
            return 0

        lax.fori_loop(0, CH, _ones, 0)

        def _zdacc(i, _):
            pltpu.sync_copy(
                zdeg_v, deg_sh.at[pl.ds(pl.multiple_of(start + i * 104, 8), 104)])
            return 0

        lax.fori_loop(0, RPS // 104, _zdacc, 0)

        @pl.when(s == NS - 1)
        def _():
            pltpu.sync_copy(zdeg_v.at[pl.ds(0, TAIL)],
                            deg_sh.at[pl.ds(RPS * NS, TAIL)])

    pltpu.sync_copy(src3.at[wid], src_v)
    pltpu.sync_copy(dst3.at[wid], dst_v)

    plsc.subcore_barrier()

    def _iter(it, _):
        base = it * K

        @pl.when(it > 0)
        def _():
            for k in range(K):
                pltpu.make_async_copy(
                    rows_v.at[k], acc_sh.at[dst_v.at[0]], ssems[k]).wait()
                if with_deg:
                    pltpu.make_async_copy(
                        ones_v, deg_sh.at[dst_v.at[0]], dsems[k]).wait()

        gd = [
            pltpu.async_copy(table.at[src_v.at[base + k]],
                             rows_v.at[k], gsems[k])
            for k in range(K)
        ]
        for k in range(K):
            gd[k].wait()
            pltpu.async_copy(
                rows_v.at[k], acc_sh.at[dst_v.at[base + k]], ssems[k],
                add=True)
            if with_deg:
                pltpu.async_copy(
                    ones_v, deg_sh.at[dst_v.at[base + k]], dsems[k],
                    add=True)
        return 0

    lax.fori_loop(0, NCH // K, _iter, 0)
    for k in range(K):
        pltpu.make_async_copy(
            rows_v.at[k], acc_sh.at[dst_v.at[0]], ssems[k]).wait()
        if with_deg:
            pltpu.make_async_copy(
                ones_v, deg_sh.at[dst_v.at[0]], dsems[k]).wait()

    plsc.subcore_barrier()

    def _share_copy(src_sh, dst_hbm):
        pltpu.sync_copy(src_sh.at[pl.ds(start, RPS)], dst_hbm.at[pl.ds(start, RPS)])

        @pl.when(s == NS - 1)
        def _():
            pltpu.sync_copy(src_sh.at[pl.ds(RPS * NS, TAIL)],
                            dst_hbm.at[pl.ds(RPS * NS, TAIL)])

    @pl.when(c == 0)
    def _():
        _share_copy(acc_sh, out0)
        if with_deg:
            _share_copy(deg_sh, dego0)

    @pl.when(c == 1)
    def _():
        _share_copy(acc_sh, out1)
        if with_deg:
            _share_copy(deg_sh, dego1)


def _make_sc(with_deg):
    f32 = jnp.float32
    bf16 = jnp.bfloat16
    outs = [jax.ShapeDtypeStruct((N, D), bf16), jax.ShapeDtypeStruct((N, D), bf16)]
    scratch = [
        pltpu.VMEM((NCH, CH), jnp.int32),
        pltpu.VMEM((NCH, CH), jnp.int32),
        pltpu.VMEM((K, CH, D), bf16),
    ]
    if with_deg:
        outs += [jax.ShapeDtypeStruct((N, 16), f32), jax.ShapeDtypeStruct((N, 16), f32)]
        scratch += [pltpu.VMEM((CH, 16), f32)]
    scratch += [pltpu.VMEM((104, D), bf16)]
    if with_deg:
        scratch += [pltpu.VMEM((104, 16), f32)]
    scratch += [pltpu.VMEM_SHARED((N, D), bf16)]
    if with_deg:
        scratch += [pltpu.VMEM_SHARED((N, 16), f32)]
    nsem = 3 * K if with_deg else 2 * K
    scratch += [pltpu.SemaphoreType.DMA] * nsem

    return pl.kernel(
        functools.partial(_sc_body, with_deg),
        out_type=tuple(outs),
        mesh=_MESH,
        scratch_types=scratch,
        compiler_params=pltpu.CompilerParams(use_tc_tiling_on_sc=False),
    )


_SC_L1 = _make_sc(True)
_SC_L2 = _make_sc(False)

_BLK = 1000


def _deg(dg0_ref, dg1_ref):
    dg = dg0_ref[:, 0:1] + dg1_ref[:, 0:1]
    return jnp.maximum(dg, 1.0)


def _tc_body1(x_ref, p0_ref, p1_ref, dg0_ref, dg1_ref, wl_ref, b_ref, wr_ref,
              ob_ref):
    agg = p0_ref[...].astype(jnp.float32) + p1_ref[...].astype(jnp.float32)
    mean = agg / _deg(dg0_ref, dg1_ref)
    dn = (((1,), (1,)), ((), ()))
    h = lax.dot_general(mean, wl_ref[...], dn, preferred_element_type=jnp.float32)
    h = h + b_ref[...] + lax.dot_general(
        x_ref[...].astype(jnp.float32), wr_ref[...], dn,
        preferred_element_type=jnp.float32)
    ob_ref[...] = jax.nn.relu(h).astype(jnp.bfloat16)


def _tc_body2(h_ref, q0_ref, q1_ref, dg0_ref, dg1_ref, wl_ref,
              b_ref, wr_ref, o_ref):
    agg = q0_ref[...].astype(jnp.float32) + q1_ref[...].astype(jnp.float32)
    mean = agg / _deg(dg0_ref, dg1_ref)
    dn = (((1,), (1,)), ((), ()))
    h = lax.dot_general(mean, wl_ref[...], dn, preferred_element_type=jnp.float32)
    h = h + b_ref[...] + lax.dot_general(
        h_ref[...].astype(jnp.float32), wr_ref[...], dn,
        preferred_element_type=jnp.float32)
    o_ref[...] = jax.nn.sigmoid(h)


_row = pl.BlockSpec((_BLK, D), lambda i: (i, 0))
_half = pl.BlockSpec((_BLK, HD), lambda i: (i, 0))
_dgs = pl.BlockSpec((_BLK, 16), lambda i: (i, 0))
_full = pl.BlockSpec((D, D), lambda i: (0, 0))
_bias = pl.BlockSpec((1, D), lambda i: (0, 0))

_rowb = pl.BlockSpec((_BLK, D), lambda i: (i, 0))

_TC_L1 = pl.pallas_call(
    _tc_body1,
    grid=(N // _BLK,),
    in_specs=[_rowb, _rowb, _rowb, _dgs, _dgs, _full, _bias, _full],
    out_specs=_rowb,
    out_shape=jax.ShapeDtypeStruct((N, D), jnp.bfloat16),
)

_TC_L2 = pl.pallas_call(
    _tc_body2,
    grid=(N // _BLK,),
    in_specs=[_rowb, _rowb, _rowb, _dgs, _dgs, _full, _bias, _full],
    out_specs=_row,
    out_shape=jax.ShapeDtypeStruct((N, D), jnp.float32),
)


def kernel(x, edge_index, W1_l, b1_l, W1_r, W2_l, b2_l, W2_r):
    src3 = edge_index[0].astype(jnp.int32).reshape(NW, NCH, CH)
    dst3 = edge_index[1].astype(jnp.int32).reshape(NW, NCH, CH)
    xb = x.astype(jnp.bfloat16)

    p0, p1, dg0, dg1 = _SC_L1(xb, src3, dst3)
    hb = _TC_L1(xb, p0, p1, dg0, dg1, W1_l, b1_l.reshape(1, D), W1_r)
    q0, q1 = _SC_L2(hb, src3, dst3)
    return _TC_L2(hb, q0, q1, dg0, dg1, W2_l, b2_l.reshape(1, D), W2_r)
